# Initial kernel scaffold; baseline (speedup 1.0000x reference)
import jax
import jax.numpy as jnp
from jax.experimental import pallas as pl


def kernel(node_features, edge_index, W_msg, b_msg, W_upd, b_upd, gamma, beta):
    raise NotImplementedError("write your pallas kernel here")


# speedup vs baseline: 4.3682x; 4.3682x over previous
import functools

import jax
import jax.numpy as jnp
from jax import lax
from jax.experimental import pallas as pl
from jax.experimental.pallas import tpu as pltpu
from jax.experimental.pallas import tpu_sc as plsc

N = 10000
E = 320000
D = 128

NC = 2
NS = 16
NW = NC * NS

EPW = E // NW
EC = 80
NCHUNK = EPW // EC
NP = 10240
RPT = NP // NS
ZR = 128


def _pq_body(x_ref, w_ref, b_ref, p_ref, q_ref):
    x = x_ref[...]
    w = w_ref[...]
    p_ref[...] = jnp.dot(x, w[:D, :], preferred_element_type=jnp.float32) + b_ref[...]
    q_ref[...] = jnp.dot(x, w[D:, :], preferred_element_type=jnp.float32)


def _pq(x, W_msg, b_msg):
    blk = 1000
    grid = N // blk
    return pl.pallas_call(
        _pq_body,
        grid=(grid,),
        in_specs=[
            pl.BlockSpec((blk, D), lambda i: (i, 0)),
            pl.BlockSpec((2 * D, D), lambda i: (0, 0)),
            pl.BlockSpec((1, D), lambda i: (0, 0)),
        ],
        out_specs=[
            pl.BlockSpec((blk, D), lambda i: (i, 0)),
            pl.BlockSpec((blk, D), lambda i: (i, 0)),
        ],
        out_shape=[
            jax.ShapeDtypeStruct((N, D), jnp.float32),
            jax.ShapeDtypeStruct((N, D), jnp.float32),
        ],
    )(x, W_msg, b_msg.reshape(1, D))


def _edge_body(p_hbm, q_hbm, row_hbm, col_hbm, out_hbm,
               row_v, col_v, p_v, q_v, z_v, agg_sh, gsem):
    cid = lax.axis_index("c")
    sid = lax.axis_index("s")
    wid = sid * NC + cid

    def _zrow(r):
        for g in range(8):
            z_v[r, pl.ds(g * 16, 16)] = jnp.zeros((16,), jnp.float32)
    pl.loop(0, ZR)(_zrow)
    for j in range(RPT // ZR):
        pltpu.sync_copy(z_v, agg_sh.at[pl.ds(sid * RPT + j * ZR, ZR)])
    plsc.subcore_barrier()

    def _chunk(t):
        base = wid * EPW + t * EC
        pltpu.sync_copy(row_hbm.at[pl.ds(base, EC)], row_v)
        pltpu.sync_copy(col_hbm.at[pl.ds(base, EC)], col_v)
        pltpu.async_copy(p_hbm.at[row_v], p_v, gsem).wait()
        pltpu.async_copy(q_hbm.at[col_v], q_v, gsem).wait()

        def _row(r):
            for g in range(8):
                s = pl.ds(g * 16, 16)
                m = p_v[r, s] + q_v[r, s]
                p_v[r, s] = jnp.maximum(m, 0.0)
        pl.loop(0, EC)(_row)
        pltpu.sync_copy(p_v, agg_sh.at[col_v], add=True)
    pl.loop(0, NCHUNK)(_chunk)

    plsc.subcore_barrier()
    pltpu.sync_copy(agg_sh.at[pl.ds(sid * RPT, RPT)],
                    out_hbm.at[cid, pl.ds(sid * RPT, RPT)])


def _edge(P, Q, row, col):
    mesh = plsc.VectorSubcoreMesh(core_axis_name="c", subcore_axis_name="s")
    f = functools.partial(
        pl.kernel,
        out_type=jax.ShapeDtypeStruct((NC, NP, D), jnp.float32),
        mesh=mesh,
        scratch_types=[
            pltpu.VMEM((EC,), jnp.int32),
            pltpu.VMEM((EC,), jnp.int32),
            pltpu.VMEM((EC, D), jnp.float32),
            pltpu.VMEM((EC, D), jnp.float32),
            pltpu.VMEM((ZR, D), jnp.float32),
            pltpu.VMEM_SHARED((NP, D), jnp.float32),
            pltpu.SemaphoreType.DMA,
        ],
    )(_edge_body)
    return f(P, Q, row, col)


def _upd_body(x_ref, a0_ref, a1_ref, w_ref, b_ref, g_ref, be_ref, o_ref):
    x = x_ref[...]
    a = a0_ref[0] + a1_ref[0]
    w = w_ref[...]
    u = (jnp.dot(x, w[:D, :], preferred_element_type=jnp.float32)
         + jnp.dot(a, w[D:, :], preferred_element_type=jnp.float32)
         + b_ref[...])
    u = jnp.maximum(u, 0.0)
    mean = jnp.mean(u, axis=1, keepdims=True)
    c = u - mean
    var = jnp.mean(c * c, axis=1, keepdims=True)
    o_ref[...] = c * lax.rsqrt(var + 1e-5) * g_ref[...] + be_ref[...]


def _upd(x, aggs, W_upd, b_upd, gamma, beta):
    blk = 1000
    grid = N // blk
    return pl.pallas_call(
        _upd_body,
        grid=(grid,),
        in_specs=[
            pl.BlockSpec((blk, D), lambda i: (i, 0)),
            pl.BlockSpec((1, blk, D), lambda i: (0, i, 0)),
            pl.BlockSpec((1, blk, D), lambda i: (1, i, 0)),
            pl.BlockSpec((2 * D, D), lambda i: (0, 0)),
            pl.BlockSpec((1, D), lambda i: (0, 0)),
            pl.BlockSpec((1, D), lambda i: (0, 0)),
            pl.BlockSpec((1, D), lambda i: (0, 0)),
        ],
        out_specs=pl.BlockSpec((blk, D), lambda i: (i, 0)),
        out_shape=jax.ShapeDtypeStruct((N, D), jnp.float32),
    )(x, aggs, aggs, W_upd, b_upd.reshape(1, D),
      gamma.reshape(1, D), beta.reshape(1, D))


def kernel(node_features, edge_index, W_msg, b_msg, W_upd, b_upd, gamma, beta):
    row = edge_index[0]
    col = edge_index[1]
    P, Q = _pq(node_features, W_msg, b_msg)
    aggs = _edge(P, Q, row, col)
    return _upd(node_features, aggs, W_upd, b_upd, gamma, beta)


# speedup vs baseline: 9.3517x; 2.1409x over previous
import functools

import jax
import jax.numpy as jnp
from jax import lax
from jax.experimental import pallas as pl
from jax.experimental.pallas import tpu as pltpu
from jax.experimental.pallas import tpu_sc as plsc

N = 10000
E = 320000
D = 128

NC = 2
NS = 16
NW = NC * NS

EPW = E // NW
EC = 40
NCHUNK = EPW // EC
NP = 10240
RPT = NP // NS
ZR = 128


def _pq_body(x_ref, w_ref, b_ref, p_ref, q_ref):
    x = x_ref[...]
    w = w_ref[...]
    p_ref[...] = jnp.dot(x, w[:D, :], preferred_element_type=jnp.float32) + b_ref[...]
    q_ref[...] = jnp.dot(x, w[D:, :], preferred_element_type=jnp.float32)


def _pq(x, W_msg, b_msg):
    blk = 1000
    grid = N // blk
    return pl.pallas_call(
        _pq_body,
        grid=(grid,),
        in_specs=[
            pl.BlockSpec((blk, D), lambda i: (i, 0)),
            pl.BlockSpec((2 * D, D), lambda i: (0, 0)),
            pl.BlockSpec((1, D), lambda i: (0, 0)),
        ],
        out_specs=[
            pl.BlockSpec((blk, D), lambda i: (i, 0)),
            pl.BlockSpec((blk, D), lambda i: (i, 0)),
        ],
        out_shape=[
            jax.ShapeDtypeStruct((N, D), jnp.float32),
            jax.ShapeDtypeStruct((N, D), jnp.float32),
        ],
    )(x, W_msg, b_msg.reshape(1, D))


def _edge_body(p_hbm, q_hbm, row_hbm, col_hbm, out_hbm,
               row_r, col_r, p0, p1, q0, q1, s0, s1, agg_sh,
               isem, psem0, psem1, ssem0, ssem1):
    cid = lax.axis_index("c")
    sid = lax.axis_index("s")
    wid = sid * NC + cid
    ebase = wid * EPW

    def _load_idx_sync(t, j):
        pltpu.sync_copy(row_hbm.at[pl.ds(ebase + t * EC, EC)], row_r.at[j])
        pltpu.sync_copy(col_hbm.at[pl.ds(ebase + t * EC, EC)], col_r.at[j])

    def _start_gather(j, pv, qv, sem):
        pltpu.async_copy(p_hbm.at[row_r.at[j]], pv, sem)
        pltpu.async_copy(q_hbm.at[col_r.at[j]], qv, sem)

    def _wait_gather(pv, qv, sem):
        pltpu.make_async_copy(p_hbm.at[row_r.at[0]], pv, sem).wait()
        pltpu.make_async_copy(q_hbm.at[col_r.at[0]], qv, sem).wait()

    _load_idx_sync(0, 0)
    _load_idx_sync(1, 1)
    _start_gather(0, p0, q0, psem0)
    _start_gather(1, p1, q1, psem1)

    def _zrow(r):
        for g in range(8):
            s0[r, pl.ds(g * 16, 16)] = jnp.zeros((16,), jnp.float32)
    pl.loop(0, EC)(_zrow)
    for j in range(RPT // EC):
        pltpu.sync_copy(s0, agg_sh.at[pl.ds(sid * RPT + j * EC, EC)])
    plsc.subcore_barrier()

    def _phase(t, pv, qv, sv, psem, ssem):
        jt = lax.rem(t, 4)
        jn = lax.rem(t + 2, 4)
        _wait_gather(pv, qv, psem)

        @pl.when(t >= 2)
        def _():
            pltpu.make_async_copy(sv, agg_sh.at[col_r.at[0]], ssem).wait()

        @pl.when(t + 2 < NCHUNK)
        def _():
            pltpu.async_copy(row_hbm.at[pl.ds(ebase + (t + 2) * EC, EC)],
                             row_r.at[jn], isem)
            pltpu.async_copy(col_hbm.at[pl.ds(ebase + (t + 2) * EC, EC)],
                             col_r.at[jn], isem)

        def _row(r):
            for g in range(8):
                s = pl.ds(g * 16, 16)
                sv[r, s] = jnp.maximum(pv[r, s] + qv[r, s], 0.0)
        pl.loop(0, EC)(_row)

        pltpu.async_copy(sv, agg_sh.at[col_r.at[jt]], ssem, add=True)

        @pl.when(t + 2 < NCHUNK)
        def _():
            pltpu.make_async_copy(row_hbm.at[pl.ds(0, EC)], row_r.at[0], isem).wait()
            pltpu.make_async_copy(col_hbm.at[pl.ds(0, EC)], col_r.at[0], isem).wait()
            _start_gather(jn, pv, qv, psem)

    def _pair(u):
        _phase(2 * u, p0, q0, s0, psem0, ssem0)
        _phase(2 * u + 1, p1, q1, s1, psem1, ssem1)
    pl.loop(0, NCHUNK // 2)(_pair)

    pltpu.make_async_copy(s0, agg_sh.at[col_r.at[0]], ssem0).wait()
    pltpu.make_async_copy(s1, agg_sh.at[col_r.at[0]], ssem1).wait()

    plsc.subcore_barrier()
    pltpu.sync_copy(agg_sh.at[pl.ds(sid * RPT, RPT)],
                    out_hbm.at[cid, pl.ds(sid * RPT, RPT)])


def _edge(P, Q, edge_index):
    mesh = plsc.VectorSubcoreMesh(core_axis_name="c", subcore_axis_name="s")
    f = functools.partial(
        pl.kernel,
        out_type=jax.ShapeDtypeStruct((NC, NP, D), jnp.float32),
        mesh=mesh,
        scratch_types=[
            pltpu.VMEM((4, EC), jnp.int32),
            pltpu.VMEM((4, EC), jnp.int32),
            pltpu.VMEM((EC, D), jnp.float32),
            pltpu.VMEM((EC, D), jnp.float32),
            pltpu.VMEM((EC, D), jnp.float32),
            pltpu.VMEM((EC, D), jnp.float32),
            pltpu.VMEM((EC, D), jnp.float32),
            pltpu.VMEM((EC, D), jnp.float32),
            pltpu.VMEM_SHARED((NP, D), jnp.float32),
            pltpu.SemaphoreType.DMA,
            pltpu.SemaphoreType.DMA,
            pltpu.SemaphoreType.DMA,
            pltpu.SemaphoreType.DMA,
            pltpu.SemaphoreType.DMA,
        ],
    )(_edge_body)
    return f(P, Q, edge_index[0], edge_index[1])


def _upd_body(x_ref, a0_ref, a1_ref, w_ref, b_ref, g_ref, be_ref, o_ref):
    x = x_ref[...]
    a = a0_ref[0] + a1_ref[0]
    w = w_ref[...]
    u = (jnp.dot(x, w[:D, :], preferred_element_type=jnp.float32)
         + jnp.dot(a, w[D:, :], preferred_element_type=jnp.float32)
         + b_ref[...])
    u = jnp.maximum(u, 0.0)
    mean = jnp.mean(u, axis=1, keepdims=True)
    c = u - mean
    var = jnp.mean(c * c, axis=1, keepdims=True)
    o_ref[...] = c * lax.rsqrt(var + 1e-5) * g_ref[...] + be_ref[...]


def _upd(x, aggs, W_upd, b_upd, gamma, beta):
    blk = 1000
    grid = N // blk
    return pl.pallas_call(
        _upd_body,
        grid=(grid,),
        in_specs=[
            pl.BlockSpec((blk, D), lambda i: (i, 0)),
            pl.BlockSpec((1, blk, D), lambda i: (0, i, 0)),
            pl.BlockSpec((1, blk, D), lambda i: (1, i, 0)),
            pl.BlockSpec((2 * D, D), lambda i: (0, 0)),
            pl.BlockSpec((1, D), lambda i: (0, 0)),
            pl.BlockSpec((1, D), lambda i: (0, 0)),
            pl.BlockSpec((1, D), lambda i: (0, 0)),
        ],
        out_specs=pl.BlockSpec((blk, D), lambda i: (i, 0)),
        out_shape=jax.ShapeDtypeStruct((N, D), jnp.float32),
    )(x, aggs, aggs, W_upd, b_upd.reshape(1, D),
      gamma.reshape(1, D), beta.reshape(1, D))


def kernel(node_features, edge_index, W_msg, b_msg, W_upd, b_upd, gamma, beta):
    P, Q = _pq(node_features, W_msg, b_msg)
    aggs = _edge(P, Q, edge_index)
    return _upd(node_features, aggs, W_upd, b_upd, gamma, beta)
